# baseline (speedup 1.0000x reference)
import functools
import math

import jax
import jax.numpy as jnp
from jax import lax
from jax.experimental import pallas as pl
from jax.experimental.pallas import tpu as pltpu
from jax.experimental.pallas import tpu_sc as plsc

D_MODEL = 768
NUM_EXPERTS = 8
D_FF = 3072
SEQ = 2048
CAPACITY = 512
LANES = 16
NUM_WORKERS = 32

BLK_F = 3072
BLK_R = 128

_MESH = plsc.VectorSubcoreMesh(core_axis_name="c", subcore_axis_name="s")
_SC_PARAMS = pltpu.CompilerParams(needs_layout_passes=False)


def _worker_id():
    return lax.axis_index("s") * 2 + lax.axis_index("c")


def _gate_body(tok_ref, gwt_ref, out_ref):
    out_ref[...] = jnp.dot(
        tok_ref[...], gwt_ref[...], preferred_element_type=jnp.float32
    )


def _gate(tokens, gate_w_t):
    return pl.pallas_call(
        _gate_body,
        out_shape=jax.ShapeDtypeStruct((SEQ, NUM_EXPERTS), jnp.float32),
    )(tokens, gate_w_t)


@functools.partial(
    pl.kernel,
    out_type=[
        jax.ShapeDtypeStruct((NUM_EXPERTS * CAPACITY,), jnp.int32),
        jax.ShapeDtypeStruct((SEQ,), jnp.int32),
        jax.ShapeDtypeStruct((SEQ,), jnp.float32),
        jax.ShapeDtypeStruct((LANES,), jnp.int32),
    ],
    mesh=_MESH,
    compiler_params=_SC_PARAMS,
    scratch_types=[
        pltpu.VMEM((NUM_EXPERTS, SEQ), jnp.float32),
        pltpu.VMEM((NUM_EXPERTS * CAPACITY,), jnp.int32),
        pltpu.VMEM((SEQ,), jnp.int32),
        pltpu.VMEM((SEQ,), jnp.float32),
        pltpu.VMEM((LANES,), jnp.int32),
    ],
)
def _route(lgt_hbm, stt_hbm, tts_hbm, scale_hbm, cnt_hbm,
           lg_v, stt_v, tts_v, scale_v, cnt_v):
    wid = _worker_id()

    @pl.when(wid == 0)
    def _():
        pltpu.sync_copy(lgt_hbm, lg_v)

        def zero_body(i, _):
            base = i * LANES
            stt_v[pl.ds(base, LANES)] = (
                base + lax.iota(jnp.int32, LANES)
            ) & (SEQ - 1)
            return 0
        lax.fori_loop(0, NUM_EXPERTS * CAPACITY // LANES, zero_body, 0)

        def body(v, counts):
            ls = [lg_v[e, pl.ds(v * LANES, LANES)] for e in range(NUM_EXPERTS)]
            m = ls[0]
            for e in range(1, NUM_EXPERTS):
                m = jnp.maximum(m, ls[e])
            eid = jnp.full((LANES,), NUM_EXPERTS - 1, jnp.int32)
            for e in range(NUM_EXPERTS - 2, -1, -1):
                eid = jnp.where(ls[e] == m, e, eid)
            den = jnp.zeros((LANES,), jnp.float32)
            for e in range(NUM_EXPERTS):
                den = den + jnp.exp(ls[e] - m)
            prob = 1.0 / den

            rank = jnp.zeros((LANES,), jnp.int32)
            new_counts = []
            for e in range(NUM_EXPERTS):
                me = eid == e
                mi = jnp.where(me, 1, 0).astype(jnp.int32)
                cs = plsc.cumsum(mi)
                rank = jnp.where(me, cs - 1 + counts[e], rank)
                new_counts.append(counts[e] + jnp.sum(mi))

            kept = rank < CAPACITY
            tok = v * LANES + lax.iota(jnp.int32, LANES)
            slot = eid * CAPACITY + rank
            slot_c = jnp.where(kept, slot, 0)
            tts_v[pl.ds(v * LANES, LANES)] = slot_c
            scale_v[pl.ds(v * LANES, LANES)] = jnp.where(kept, prob, 0.0)
            plsc.store_scatter(stt_v, [slot_c], tok, mask=kept)
            return tuple(new_counts)

        counts = lax.fori_loop(
            0, SEQ // LANES, body, (jnp.int32(0),) * NUM_EXPERTS
        )

        cv = jnp.zeros((LANES,), jnp.int32)
        lane = lax.iota(jnp.int32, LANES)
        for e in range(NUM_EXPERTS):
            cv = jnp.where(lane == e, jnp.minimum(counts[e], CAPACITY), cv)
        cnt_v[...] = cv

        pltpu.sync_copy(stt_v, stt_hbm)
        pltpu.sync_copy(tts_v, tts_hbm)
        pltpu.sync_copy(scale_v, scale_hbm)
        pltpu.sync_copy(cnt_v, cnt_hbm)


_ROWS_PER_W = NUM_EXPERTS * CAPACITY // NUM_WORKERS


@functools.partial(
    pl.kernel,
    out_type=jax.ShapeDtypeStruct((NUM_EXPERTS * CAPACITY, D_MODEL), jnp.float32),
    mesh=_MESH,
    compiler_params=_SC_PARAMS,
    scratch_types=[
        pltpu.VMEM((_ROWS_PER_W,), jnp.int32),
        pltpu.VMEM((_ROWS_PER_W, D_MODEL), jnp.float32),
        pltpu.SemaphoreType.DMA,
    ],
)
def _dispatch(tok_hbm, stt_hbm, out_hbm, idx_v, rows_v, sem):
    base = _worker_id() * _ROWS_PER_W
    pltpu.sync_copy(stt_hbm.at[pl.ds(base, _ROWS_PER_W)], idx_v)
    pltpu.async_copy(tok_hbm.at[idx_v], rows_v, sem).wait()
    pltpu.sync_copy(rows_v, out_hbm.at[pl.ds(base, _ROWS_PER_W)])


def _ffn_body(cnt_ref, x_ref, w1_ref, w2_ref, out_ref, w1b_ref, w2b_ref):
    e = pl.program_id(0)
    rb = pl.program_id(1)

    @pl.when(rb == 0)
    def _():
        w1b_ref[...] = w1_ref[0].astype(jnp.bfloat16)
        w2b_ref[...] = w2_ref[0].astype(jnp.bfloat16)

    active = rb * BLK_R < cnt_ref[e]

    @pl.when(active)
    def _():
        x = x_ref[...].astype(jnp.bfloat16)
        h = jax.nn.gelu(
            jnp.dot(x, w1b_ref[...], preferred_element_type=jnp.float32)
        )
        out_ref[...] = jnp.dot(
            h.astype(jnp.bfloat16), w2b_ref[...],
            preferred_element_type=jnp.float32,
        )

    @pl.when(jnp.logical_not(active))
    def _():
        out_ref[...] = jnp.zeros_like(out_ref)


def _ffn(counts, dispatch, w1, w2):
    grid_spec = pltpu.PrefetchScalarGridSpec(
        num_scalar_prefetch=1,
        grid=(NUM_EXPERTS, CAPACITY // BLK_R),
        in_specs=[
            pl.BlockSpec(
                (BLK_R, D_MODEL),
                lambda e, rb, *_: (e * (CAPACITY // BLK_R) + rb, 0),
            ),
            pl.BlockSpec((1, D_MODEL, D_FF), lambda e, rb, *_: (e, 0, 0)),
            pl.BlockSpec((1, D_FF, D_MODEL), lambda e, rb, *_: (e, 0, 0)),
        ],
        out_specs=pl.BlockSpec(
            (BLK_R, D_MODEL),
            lambda e, rb, *_: (e * (CAPACITY // BLK_R) + rb, 0),
        ),
        scratch_shapes=[
            pltpu.VMEM((D_MODEL, D_FF), jnp.bfloat16),
            pltpu.VMEM((D_FF, D_MODEL), jnp.bfloat16),
        ],
    )
    return pl.pallas_call(
        _ffn_body,
        grid_spec=grid_spec,
        out_shape=jax.ShapeDtypeStruct(
            (NUM_EXPERTS * CAPACITY, D_MODEL), jnp.float32
        ),
        compiler_params=pltpu.CompilerParams(
            dimension_semantics=("arbitrary", "arbitrary"),
        ),
    )(counts, dispatch, w1, w2)


_TOKS_PER_W = SEQ // NUM_WORKERS


@functools.partial(
    pl.kernel,
    out_type=jax.ShapeDtypeStruct((SEQ, D_MODEL), jnp.float32),
    mesh=_MESH,
    compiler_params=_SC_PARAMS,
    scratch_types=[
        pltpu.VMEM((_TOKS_PER_W,), jnp.int32),
        pltpu.VMEM((_TOKS_PER_W,), jnp.float32),
        pltpu.VMEM((_TOKS_PER_W, D_MODEL), jnp.float32),
        pltpu.SemaphoreType.DMA,
    ],
)
def _combine(y_hbm, tts_hbm, scale_hbm, out_hbm, idx_v, sc_v, rows_v, sem):
    base = _worker_id() * _TOKS_PER_W
    pltpu.sync_copy(tts_hbm.at[pl.ds(base, _TOKS_PER_W)], idx_v)
    pltpu.sync_copy(scale_hbm.at[pl.ds(base, _TOKS_PER_W)], sc_v)
    pltpu.async_copy(y_hbm.at[idx_v], rows_v, sem).wait()

    def body(i, _):
        s = plsc.load_gather(sc_v, [jnp.zeros((LANES,), jnp.int32) + i])
        for j in range(D_MODEL // LANES):
            rows_v[i, pl.ds(j * LANES, LANES)] = (
                rows_v[i, pl.ds(j * LANES, LANES)] * s
            )
        return 0

    lax.fori_loop(0, _TOKS_PER_W, body, 0)
    pltpu.sync_copy(rows_v, out_hbm.at[pl.ds(base, _TOKS_PER_W)])


def kernel(inputs, gate_w, w1, w2):
    tokens = inputs.reshape(-1, D_MODEL)
    logits = _gate(tokens, gate_w.T)
    stt, tts, scale, counts = _route(logits.T)
    dispatch = _dispatch(tokens, stt)
    y = _ffn(counts, dispatch, w1, w2)
    out = _combine(y, tts, scale)
    return out.reshape(inputs.shape)


# speedup vs baseline: 1.3522x; 1.3522x over previous
import functools
import math

import jax
import jax.numpy as jnp
from jax import lax
from jax.experimental import pallas as pl
from jax.experimental.pallas import tpu as pltpu
from jax.experimental.pallas import tpu_sc as plsc

D_MODEL = 768
NUM_EXPERTS = 8
D_FF = 3072
SEQ = 2048
CAPACITY = 512
LANES = 16
NUM_WORKERS = 32

BLK_F = 1536
BLK_R = 128

_MESH = plsc.VectorSubcoreMesh(core_axis_name="c", subcore_axis_name="s")
_SC_PARAMS = pltpu.CompilerParams(needs_layout_passes=False)


def _worker_id():
    return lax.axis_index("s") * 2 + lax.axis_index("c")


def _gate_body(gw_ref, tok_ref, out_ref):
    out_ref[...] = lax.dot_general(
        gw_ref[...], tok_ref[...],
        dimension_numbers=(((1,), (1,)), ((), ())),
        preferred_element_type=jnp.float32,
    )


def _gate(tokens, gate_w):
    return pl.pallas_call(
        _gate_body,
        out_shape=jax.ShapeDtypeStruct((NUM_EXPERTS, SEQ), jnp.float32),
    )(gate_w, tokens)


@functools.partial(
    pl.kernel,
    out_type=[
        jax.ShapeDtypeStruct((NUM_EXPERTS * CAPACITY,), jnp.int32),
        jax.ShapeDtypeStruct((SEQ,), jnp.int32),
        jax.ShapeDtypeStruct((SEQ,), jnp.float32),
        jax.ShapeDtypeStruct((LANES,), jnp.int32),
    ],
    mesh=_MESH,
    compiler_params=_SC_PARAMS,
    scratch_types=[
        pltpu.VMEM((NUM_EXPERTS, SEQ), jnp.float32),
        pltpu.VMEM((NUM_EXPERTS * CAPACITY,), jnp.int32),
        pltpu.VMEM((SEQ,), jnp.int32),
        pltpu.VMEM((SEQ,), jnp.float32),
        pltpu.VMEM((LANES,), jnp.int32),
    ],
)
def _route(lgt_hbm, stt_hbm, tts_hbm, scale_hbm, cnt_hbm,
           lg_v, stt_v, tts_v, scale_v, cnt_v):
    wid = _worker_id()

    @pl.when(wid == 0)
    def _():
        pltpu.sync_copy(lgt_hbm, lg_v)

        def zero_body(i, _):
            base = i * LANES
            stt_v[pl.ds(base, LANES)] = (
                base + lax.iota(jnp.int32, LANES)
            ) & (SEQ - 1)
            return 0
        lax.fori_loop(0, NUM_EXPERTS * CAPACITY // LANES, zero_body, 0)

        def body(v, counts):
            ls = [lg_v[e, pl.ds(v * LANES, LANES)] for e in range(NUM_EXPERTS)]
            m = ls[0]
            for e in range(1, NUM_EXPERTS):
                m = jnp.maximum(m, ls[e])
            eid = jnp.full((LANES,), NUM_EXPERTS - 1, jnp.int32)
            for e in range(NUM_EXPERTS - 2, -1, -1):
                eid = jnp.where(ls[e] == m, e, eid)
            den = jnp.zeros((LANES,), jnp.float32)
            for e in range(NUM_EXPERTS):
                den = den + jnp.exp(ls[e] - m)
            prob = 1.0 / den

            rank = jnp.zeros((LANES,), jnp.int32)
            new_counts = []
            for e in range(NUM_EXPERTS):
                me = eid == e
                mi = jnp.where(me, 1, 0).astype(jnp.int32)
                cs = plsc.cumsum(mi)
                rank = jnp.where(me, cs - 1 + counts[e], rank)
                new_counts.append(counts[e] + jnp.sum(mi))

            kept = rank < CAPACITY
            tok = v * LANES + lax.iota(jnp.int32, LANES)
            slot = eid * CAPACITY + rank
            slot_c = jnp.where(kept, slot, 0)
            tts_v[pl.ds(v * LANES, LANES)] = slot_c
            scale_v[pl.ds(v * LANES, LANES)] = jnp.where(kept, prob, 0.0)
            plsc.store_scatter(stt_v, [slot_c], tok, mask=kept)
            return tuple(new_counts)

        counts = lax.fori_loop(
            0, SEQ // LANES, body, (jnp.int32(0),) * NUM_EXPERTS
        )

        cv = jnp.zeros((LANES,), jnp.int32)
        lane = lax.iota(jnp.int32, LANES)
        for e in range(NUM_EXPERTS):
            cv = jnp.where(lane == e, jnp.minimum(counts[e], CAPACITY), cv)
        cnt_v[...] = cv

        pltpu.sync_copy(stt_v, stt_hbm)
        pltpu.sync_copy(tts_v, tts_hbm)
        pltpu.sync_copy(scale_v, scale_hbm)
        pltpu.sync_copy(cnt_v, cnt_hbm)


_ROWS_PER_W = NUM_EXPERTS * CAPACITY // NUM_WORKERS


@functools.partial(
    pl.kernel,
    out_type=jax.ShapeDtypeStruct((NUM_EXPERTS * CAPACITY, D_MODEL), jnp.float32),
    mesh=_MESH,
    compiler_params=_SC_PARAMS,
    scratch_types=[
        pltpu.VMEM((_ROWS_PER_W,), jnp.int32),
        pltpu.VMEM((_ROWS_PER_W, D_MODEL), jnp.float32),
        pltpu.SemaphoreType.DMA,
    ],
)
def _dispatch(tok_hbm, stt_hbm, out_hbm, idx_v, rows_v, sem):
    base = _worker_id() * _ROWS_PER_W
    pltpu.sync_copy(stt_hbm.at[pl.ds(base, _ROWS_PER_W)], idx_v)
    pltpu.async_copy(tok_hbm.at[idx_v], rows_v, sem).wait()
    pltpu.sync_copy(rows_v, out_hbm.at[pl.ds(base, _ROWS_PER_W)])


def _ffn_body(cnt_ref, x_ref, w1_ref, w2_ref, out_ref):
    e = pl.program_id(0)
    fb = pl.program_id(1)
    nblk = (cnt_ref[e] + BLK_R - 1) // BLK_R

    @pl.when(fb == 0)
    def _():
        out_ref[...] = jnp.zeros_like(out_ref)

    w1b = w1_ref[0].astype(jnp.bfloat16)
    w2b = w2_ref[0].astype(jnp.bfloat16)

    def body(rb, _):
        r0 = pl.multiple_of(rb * BLK_R, BLK_R)
        x = x_ref[pl.ds(r0, BLK_R), :].astype(jnp.bfloat16)
        h = jax.nn.gelu(
            jnp.dot(x, w1b, preferred_element_type=jnp.float32)
        )
        out_ref[pl.ds(r0, BLK_R), :] += jnp.dot(
            h.astype(jnp.bfloat16), w2b, preferred_element_type=jnp.float32
        )
        return 0

    lax.fori_loop(0, nblk, body, 0)


def _ffn(counts, dispatch, w1, w2):
    grid_spec = pltpu.PrefetchScalarGridSpec(
        num_scalar_prefetch=1,
        grid=(NUM_EXPERTS, D_FF // BLK_F),
        in_specs=[
            pl.BlockSpec((CAPACITY, D_MODEL), lambda e, fb, *_: (e, 0)),
            pl.BlockSpec((1, D_MODEL, BLK_F), lambda e, fb, *_: (e, 0, fb)),
            pl.BlockSpec((1, BLK_F, D_MODEL), lambda e, fb, *_: (e, fb, 0)),
        ],
        out_specs=pl.BlockSpec((CAPACITY, D_MODEL), lambda e, fb, *_: (e, 0)),
    )
    return pl.pallas_call(
        _ffn_body,
        grid_spec=grid_spec,
        out_shape=jax.ShapeDtypeStruct(
            (NUM_EXPERTS * CAPACITY, D_MODEL), jnp.float32
        ),
        compiler_params=pltpu.CompilerParams(
            dimension_semantics=("arbitrary", "arbitrary"),
        ),
    )(counts, dispatch, w1, w2)


_TOKS_PER_W = SEQ // NUM_WORKERS


@functools.partial(
    pl.kernel,
    out_type=jax.ShapeDtypeStruct((SEQ, D_MODEL), jnp.float32),
    mesh=_MESH,
    compiler_params=_SC_PARAMS,
    scratch_types=[
        pltpu.VMEM((_TOKS_PER_W,), jnp.int32),
        pltpu.VMEM((_TOKS_PER_W,), jnp.float32),
        pltpu.VMEM((_TOKS_PER_W, D_MODEL), jnp.float32),
        pltpu.SemaphoreType.DMA,
    ],
)
def _combine(y_hbm, tts_hbm, scale_hbm, out_hbm, idx_v, sc_v, rows_v, sem):
    base = _worker_id() * _TOKS_PER_W
    pltpu.sync_copy(tts_hbm.at[pl.ds(base, _TOKS_PER_W)], idx_v)
    pltpu.sync_copy(scale_hbm.at[pl.ds(base, _TOKS_PER_W)], sc_v)
    pltpu.async_copy(y_hbm.at[idx_v], rows_v, sem).wait()

    def body(i, _):
        s = plsc.load_gather(sc_v, [jnp.zeros((LANES,), jnp.int32) + i])
        for j in range(D_MODEL // LANES):
            rows_v[i, pl.ds(j * LANES, LANES)] = (
                rows_v[i, pl.ds(j * LANES, LANES)] * s
            )
        return 0

    lax.fori_loop(0, _TOKS_PER_W, body, 0)
    pltpu.sync_copy(rows_v, out_hbm.at[pl.ds(base, _TOKS_PER_W)])


def kernel(inputs, gate_w, w1, w2):
    tokens = inputs.reshape(-1, D_MODEL)
    logits_t = _gate(tokens, gate_w)
    stt, tts, scale, counts = _route(logits_t)
    dispatch = _dispatch(tokens, stt)
    y = _ffn(counts, dispatch, w1, w2)
    out = _combine(y, tts, scale)
    return out.reshape(inputs.shape)
